# baseline (speedup 1.0000x reference)
import functools

import jax
import jax.numpy as jnp
from jax import lax
from jax.experimental import pallas as pl
from jax.experimental.pallas import tpu as pltpu
from jax.experimental.pallas import tpu_sc as plsc

NUM_WORKERS = 32
LANES = 16
CH = 4


def _build_sc_kernel(B, S, D, NB):
    SP = S + 1
    P = ((SP + 7) // 8) * 8
    S_PER_W = S // NUM_WORKERS
    T = S_PER_W // CH
    GI = B * CH
    SLAB = CH * B * D
    mesh = plsc.VectorSubcoreMesh(core_axis_name="c", subcore_axis_name="s")

    @functools.partial(
        pl.kernel,
        mesh=mesh,
        out_type=jax.ShapeDtypeStruct((SP * B * D,), jnp.float32),
        scratch_types=[
            pltpu.VMEM((T * GI + 8,), jnp.int32),
            pltpu.VMEM((T * 8 + 8,), jnp.int32),
            pltpu.VMEM((GI, D), jnp.float32),
            pltpu.VMEM((GI, D), jnp.float32),
            pltpu.VMEM((CH, D), jnp.float32),
            pltpu.VMEM((CH, D), jnp.float32),
            pltpu.VMEM((SLAB,), jnp.float32),
            pltpu.VMEM((SLAB,), jnp.float32),
            pltpu.VMEM((SLAB,), jnp.float32),
            pltpu.VMEM((SLAB,), jnp.float32),
            pltpu.VMEM((D,), jnp.float32),
            pltpu.SemaphoreType.DMA,
            pltpu.SemaphoreType.DMA,
            pltpu.SemaphoreType.DMA,
            pltpu.SemaphoreType.DMA,
            pltpu.SemaphoreType.DMA,
            pltpu.SemaphoreType.DMA,
            pltpu.SemaphoreType.DMA,
            pltpu.SemaphoreType.DMA,
        ],
    )
    def sc_embed(gidx_hbm, pidx_hbm, table_hbm, pos_hbm, cls_hbm, out_hbm,
                 gidx_v, pidx_v, rows0, rows1, posb0, posb1,
                 slab0, slab1, slab2, slab3,
                 cls_v, sg0, sg1, sp0, sp1, so0, so1, so2, so3):
        wid = lax.axis_index("c") * 16 + lax.axis_index("s")
        s0 = wid * S_PER_W
        rows = (rows0, rows1)
        posb = (posb0, posb1)
        slab = (slab0, slab1, slab2, slab3)
        sgs = (sg0, sg1)
        sps = (sp0, sp1)
        sos = (so0, so1, so2, so3)

        pltpu.sync_copy(gidx_hbm.at[pl.ds(wid * T * GI, T * GI)],
                        gidx_v.at[pl.ds(0, T * GI)])
        pltpu.sync_copy(pidx_hbm.at[pl.ds(wid * T * 8, T * 8)],
                        pidx_v.at[pl.ds(0, T * 8)])

        @pl.when(wid == 0)
        def _():
            pltpu.sync_copy(cls_hbm, cls_v)

        def gather_start(t, k):
            pltpu.async_copy(
                table_hbm.at[gidx_v.at[pl.ds(t * GI, GI)]], rows[k], sgs[k])
            pltpu.async_copy(
                pos_hbm.at[pidx_v.at[pl.ds(t * 8, CH)]], posb[k], sps[k])

        def gather_wait(k):
            pltpu.make_async_copy(table_hbm.at[pl.ds(0, GI)],
                                  rows[k], sgs[k]).wait()
            pltpu.make_async_copy(pos_hbm.at[pl.ds(0, CH)],
                                  posb[k], sps[k]).wait()

        def out_start(t, k):
            off = (s0 + t * CH) * B * D
            pltpu.async_copy(slab[k], out_hbm.at[pl.ds(off, SLAB)], sos[k])

        def out_wait(k):
            pltpu.make_async_copy(slab[k], out_hbm.at[pl.ds(0, SLAB)],
                                  sos[k]).wait()

        def add_interleave(rk, sk):
            @pl.loop(0, NB)
            def _(lb):
                lo = lb * 128
                so = lb * (B * 128)
                for r in range(CH):
                    for v in range(128 // LANES):
                        pv = posb[rk][r, pl.ds(lo + v * LANES, LANES)]
                        for b in range(B):
                            o = so + r * B * D + b * 128 + v * LANES
                            slab[sk][pl.ds(o, LANES)] = (
                                rows[rk][b * CH + r,
                                         pl.ds(lo + v * LANES, LANES)] + pv)

        gather_start(0, 0)

        @pl.loop(0, T, step=4)
        def _(tt):
            for kk in range(4):
                t = tt + kk
                rk = kk % 2

                @pl.when(t + 1 < T)
                def _():
                    gather_start(t + 1, 1 - rk)

                gather_wait(rk)

                @pl.when(t >= 4)
                def _():
                    out_wait(kk)

                add_interleave(rk, kk)

                if kk == 0:
                    @pl.when((wid == 0) & (t == 0))
                    def _():
                        @pl.loop(0, NB)
                        def _(lb):
                            so = lb * (B * 128)
                            for v in range(128 // LANES):
                                cv = cls_v[pl.ds(lb * 128 + v * LANES,
                                                 LANES)]
                                for b in range(B):
                                    o = so + b * 128 + v * LANES
                                    slab[kk][pl.ds(o, LANES)] = cv

                out_start(t, kk)

        for k in range(4):
            out_wait(k)

        @pl.when(wid == NUM_WORKERS - 1)
        def _():
            pltpu.sync_copy(gidx_hbm.at[pl.ds(NUM_WORKERS * T * GI, 8)],
                            gidx_v.at[pl.ds(0, 8)])
            pltpu.async_copy(table_hbm.at[gidx_v.at[pl.ds(0, 8)]],
                             rows0.at[pl.ds(0, 8)], sg0).wait()
            pltpu.sync_copy(pidx_hbm.at[pl.ds(NUM_WORKERS * T * 8, 8)],
                            pidx_v.at[pl.ds(0, 8)])
            pltpu.async_copy(pos_hbm.at[pidx_v.at[pl.ds(0, CH)]],
                             posb0, sp0).wait()
            for lb in range(NB):
                for v in range(128 // LANES):
                    l = lb * 128 + v * LANES
                    pv = posb0[0, pl.ds(l, LANES)]
                    for b in range(B):
                        o = lb * B * 128 + b * 128 + v * LANES
                        slab0[pl.ds(o, LANES)] = (
                            rows0[b, pl.ds(l, LANES)] + pv)
            pltpu.sync_copy(slab0.at[pl.ds(0, B * D)],
                            out_hbm.at[pl.ds(S * B * D, B * D)])

    return sc_embed


def kernel(input, table, pos_embeds, cls):
    B, S = input.shape
    D = table.shape[1]
    SP = S + 1
    P = ((SP + 7) // 8) * 8
    NB = D // 128
    S_PER_W = S // NUM_WORKERS
    sidx = jnp.zeros((B, SP), jnp.int32).at[:, 1:].set(input)
    gmain = (sidx[:, :S]
             .reshape(B, NUM_WORKERS, S_PER_W // CH, CH)
             .transpose(1, 2, 0, 3)
             .reshape(-1))
    gtail = jnp.concatenate([sidx[:, S], jnp.zeros((8 - B,), jnp.int32)])
    gidx = jnp.concatenate([gmain, gtail])
    n_items = S // CH
    pmain = jnp.clip(
        jnp.arange(n_items, dtype=jnp.int32)[:, None] * CH
        + jnp.arange(8, dtype=jnp.int32)[None, :] - 1,
        0, S - 1).reshape(-1)
    ptail = jnp.full((8,), S - 1, jnp.int32)
    pidx = jnp.concatenate([pmain, ptail])
    pos2d = pos_embeds.reshape(S, D)
    cls1d = cls.reshape(D)
    sc = _build_sc_kernel(B, S, D, NB)
    out_flat = sc(gidx, pidx, table, pos2d, cls1d)
    return (out_flat.reshape(SP, NB, B, 128)
            .transpose(2, 0, 1, 3)
            .reshape(B, SP, D))


# speedup vs baseline: 1.5157x; 1.5157x over previous
import functools

import jax
import jax.numpy as jnp
from jax import lax
from jax.experimental import pallas as pl
from jax.experimental.pallas import tpu as pltpu
from jax.experimental.pallas import tpu_sc as plsc

NUM_WORKERS = 32
LANES = 16
CH = 4


def _build_sc_kernel(B, S, D, NB):
    SP = S + 1
    P = ((SP + 7) // 8) * 8
    S_PER_W = S // NUM_WORKERS
    T = S_PER_W // CH
    GI = B * CH
    SLAB = CH * B * D
    mesh = plsc.VectorSubcoreMesh(core_axis_name="c", subcore_axis_name="s")

    @functools.partial(
        pl.kernel,
        mesh=mesh,
        out_type=jax.ShapeDtypeStruct((SP * B * D,), jnp.float32),
        scratch_types=[
            pltpu.VMEM((T * GI + 8,), jnp.int32),
            pltpu.VMEM((T * 8 + 8,), jnp.int32),
            pltpu.VMEM((GI, D), jnp.float32),
            pltpu.VMEM((GI, D), jnp.float32),
            pltpu.VMEM((CH, D), jnp.float32),
            pltpu.VMEM((CH, D), jnp.float32),
            pltpu.VMEM((SLAB,), jnp.float32),
            pltpu.VMEM((SLAB,), jnp.float32),
            pltpu.VMEM((SLAB,), jnp.float32),
            pltpu.VMEM((SLAB,), jnp.float32),
            pltpu.VMEM((D,), jnp.float32),
            pltpu.SemaphoreType.DMA,
            pltpu.SemaphoreType.DMA,
            pltpu.SemaphoreType.DMA,
            pltpu.SemaphoreType.DMA,
            pltpu.SemaphoreType.DMA,
            pltpu.SemaphoreType.DMA,
            pltpu.SemaphoreType.DMA,
            pltpu.SemaphoreType.DMA,
        ],
    )
    def sc_embed(gidx_hbm, pidx_hbm, table_hbm, pos_hbm, cls_hbm, out_hbm,
                 gidx_v, pidx_v, rows0, rows1, posb0, posb1,
                 slab0, slab1, slab2, slab3,
                 cls_v, sg0, sg1, sp0, sp1, so0, so1, so2, so3):
        wid = lax.axis_index("c") * 16 + lax.axis_index("s")
        s0 = wid * S_PER_W
        rows = (rows0, rows1)
        posb = (posb0, posb1)
        slab = (slab0, slab1, slab2, slab3)
        sgs = (sg0, sg1)
        sps = (sp0, sp1)
        sos = (so0, so1, so2, so3)

        pltpu.sync_copy(gidx_hbm.at[pl.ds(wid * T * GI, T * GI)],
                        gidx_v.at[pl.ds(0, T * GI)])
        pltpu.sync_copy(pidx_hbm.at[pl.ds(wid * T * 8, T * 8)],
                        pidx_v.at[pl.ds(0, T * 8)])

        @pl.when(wid == 0)
        def _():
            pltpu.sync_copy(cls_hbm, cls_v)

        def gather_start(t, k):
            pltpu.async_copy(
                table_hbm.at[gidx_v.at[pl.ds(t * GI, GI)]], rows[k], sgs[k])
            pltpu.async_copy(
                pos_hbm.at[pidx_v.at[pl.ds(t * 8, CH)]], posb[k], sps[k])

        def gather_wait(k):
            pltpu.make_async_copy(table_hbm.at[pl.ds(0, GI)],
                                  rows[k], sgs[k]).wait()
            pltpu.make_async_copy(pos_hbm.at[pl.ds(0, CH)],
                                  posb[k], sps[k]).wait()

        def out_start(t, k):
            off = (s0 + t * CH) * B * D
            pltpu.async_copy(slab[k], out_hbm.at[pl.ds(off, SLAB)], sos[k])

        def out_wait(k):
            pltpu.make_async_copy(slab[k], out_hbm.at[pl.ds(0, SLAB)],
                                  sos[k]).wait()

        def add_interleave(rk, sk):
            for r in range(CH):
                for lb in range(NB):
                    for v in range(128 // LANES):
                        l = lb * 128 + v * LANES
                        pv = posb[rk][r, pl.ds(l, LANES)]
                        for b in range(B):
                            o = r * B * D + lb * B * 128 + b * 128 + v * LANES
                            slab[sk][pl.ds(o, LANES)] = (
                                rows[rk][b * CH + r, pl.ds(l, LANES)] + pv)

        gather_start(0, 0)

        @pl.loop(0, T, step=4)
        def _(tt):
            for kk in range(4):
                t = tt + kk
                rk = kk % 2

                @pl.when(t + 1 < T)
                def _():
                    gather_start(t + 1, 1 - rk)

                gather_wait(rk)

                @pl.when(t >= 4)
                def _():
                    out_wait(kk)

                add_interleave(rk, kk)

                if kk == 0:
                    @pl.when((wid == 0) & (t == 0))
                    def _():
                        for lb in range(NB):
                            for v in range(128 // LANES):
                                l = lb * 128 + v * LANES
                                cv = cls_v[pl.ds(l, LANES)]
                                for b in range(B):
                                    o = lb * B * 128 + b * 128 + v * LANES
                                    slab[kk][pl.ds(o, LANES)] = cv

                out_start(t, kk)

        for k in range(4):
            out_wait(k)

        @pl.when(wid == NUM_WORKERS - 1)
        def _():
            pltpu.sync_copy(gidx_hbm.at[pl.ds(NUM_WORKERS * T * GI, 8)],
                            gidx_v.at[pl.ds(0, 8)])
            pltpu.async_copy(table_hbm.at[gidx_v.at[pl.ds(0, 8)]],
                             rows0.at[pl.ds(0, 8)], sg0).wait()
            pltpu.sync_copy(pidx_hbm.at[pl.ds(NUM_WORKERS * T * 8, 8)],
                            pidx_v.at[pl.ds(0, 8)])
            pltpu.async_copy(pos_hbm.at[pidx_v.at[pl.ds(0, CH)]],
                             posb0, sp0).wait()
            for lb in range(NB):
                for v in range(128 // LANES):
                    l = lb * 128 + v * LANES
                    pv = posb0[0, pl.ds(l, LANES)]
                    for b in range(B):
                        o = lb * B * 128 + b * 128 + v * LANES
                        slab0[pl.ds(o, LANES)] = (
                            rows0[b, pl.ds(l, LANES)] + pv)
            pltpu.sync_copy(slab0.at[pl.ds(0, B * D)],
                            out_hbm.at[pl.ds(S * B * D, B * D)])

    return sc_embed


def kernel(input, table, pos_embeds, cls):
    B, S = input.shape
    D = table.shape[1]
    SP = S + 1
    P = ((SP + 7) // 8) * 8
    NB = D // 128
    S_PER_W = S // NUM_WORKERS
    sidx = jnp.zeros((B, SP), jnp.int32).at[:, 1:].set(input)
    gmain = (sidx[:, :S]
             .reshape(B, NUM_WORKERS, S_PER_W // CH, CH)
             .transpose(1, 2, 0, 3)
             .reshape(-1))
    gtail = jnp.concatenate([sidx[:, S], jnp.zeros((8 - B,), jnp.int32)])
    gidx = jnp.concatenate([gmain, gtail])
    n_items = S // CH
    pmain = jnp.clip(
        jnp.arange(n_items, dtype=jnp.int32)[:, None] * CH
        + jnp.arange(8, dtype=jnp.int32)[None, :] - 1,
        0, S - 1).reshape(-1)
    ptail = jnp.full((8,), S - 1, jnp.int32)
    pidx = jnp.concatenate([pmain, ptail])
    pos2d = pos_embeds.reshape(S, D)
    cls1d = cls.reshape(D)
    sc = _build_sc_kernel(B, S, D, NB)
    out_flat = sc(gidx, pidx, table, pos2d, cls1d)
    return (out_flat.reshape(SP, NB, B, 128)
            .transpose(2, 0, 1, 3)
            .reshape(B, SP, D))


# speedup vs baseline: 1.5345x; 1.0124x over previous
import functools

import jax
import jax.numpy as jnp
from jax import lax
from jax.experimental import pallas as pl
from jax.experimental.pallas import tpu as pltpu
from jax.experimental.pallas import tpu_sc as plsc

NUM_WORKERS = 32
LANES = 16
CH = 8
RH = 4


def _build_sc_kernel(B, S, D, NB):
    SP = S + 1
    S_PER_W = S // NUM_WORKERS
    T = S_PER_W // CH
    GI = B * CH
    SLABH = RH * B * D
    mesh = plsc.VectorSubcoreMesh(core_axis_name="c", subcore_axis_name="s")

    @functools.partial(
        pl.kernel,
        mesh=mesh,
        out_type=jax.ShapeDtypeStruct((SP * B * D,), jnp.float32),
        scratch_types=[
            pltpu.VMEM((T * GI + 8,), jnp.int32),
            pltpu.VMEM((T * 8 + 8,), jnp.int32),
            pltpu.VMEM((GI, D), jnp.float32),
            pltpu.VMEM((GI, D), jnp.float32),
            pltpu.VMEM((CH, D), jnp.float32),
            pltpu.VMEM((CH, D), jnp.float32),
            pltpu.VMEM((SLABH,), jnp.float32),
            pltpu.VMEM((SLABH,), jnp.float32),
            pltpu.VMEM((D,), jnp.float32),
            pltpu.SemaphoreType.DMA,
            pltpu.SemaphoreType.DMA,
            pltpu.SemaphoreType.DMA,
            pltpu.SemaphoreType.DMA,
            pltpu.SemaphoreType.DMA,
            pltpu.SemaphoreType.DMA,
        ],
    )
    def sc_embed(gidx_hbm, pidx_hbm, table_hbm, pos_hbm, cls_hbm, out_hbm,
                 gidx_v, pidx_v, rows0, rows1, posb0, posb1, slab0, slab1,
                 cls_v, sg0, sg1, sp0, sp1, so0, so1):
        wid = lax.axis_index("c") * 16 + lax.axis_index("s")
        s0 = wid * S_PER_W
        rows = (rows0, rows1)
        posb = (posb0, posb1)
        slab = (slab0, slab1)
        sgs = (sg0, sg1)
        sps = (sp0, sp1)
        sos = (so0, so1)

        pltpu.sync_copy(gidx_hbm.at[pl.ds(wid * T * GI, T * GI)],
                        gidx_v.at[pl.ds(0, T * GI)])
        pltpu.sync_copy(pidx_hbm.at[pl.ds(wid * T * 8, T * 8)],
                        pidx_v.at[pl.ds(0, T * 8)])

        @pl.when(wid == 0)
        def _():
            pltpu.sync_copy(cls_hbm, cls_v)

        def gather_start(t, k):
            pltpu.async_copy(
                table_hbm.at[gidx_v.at[pl.ds(t * GI, GI)]], rows[k], sgs[k])
            pltpu.async_copy(
                pos_hbm.at[pidx_v.at[pl.ds(t * 8, CH)]], posb[k], sps[k])

        def gather_wait(k):
            pltpu.make_async_copy(table_hbm.at[pl.ds(0, GI)],
                                  rows[k], sgs[k]).wait()
            pltpu.make_async_copy(pos_hbm.at[pl.ds(0, CH)],
                                  posb[k], sps[k]).wait()

        def out_start(t, h):
            off = (s0 + t * CH + h * RH) * B * D
            pltpu.async_copy(slab[h], out_hbm.at[pl.ds(off, SLABH)], sos[h])

        def out_wait(h):
            pltpu.make_async_copy(slab[h], out_hbm.at[pl.ds(0, SLABH)],
                                  sos[h]).wait()

        def add_interleave(rk, h):
            for rl in range(RH):
                r = h * RH + rl
                for lb in range(NB):
                    for v in range(128 // LANES):
                        l = lb * 128 + v * LANES
                        pv = posb[rk][r, pl.ds(l, LANES)]
                        for b in range(B):
                            o = (rl * B * D + lb * B * 128 + b * 128
                                 + v * LANES)
                            slab[h][pl.ds(o, LANES)] = (
                                rows[rk][b * CH + r, pl.ds(l, LANES)] + pv)

        gather_start(0, 0)

        @pl.loop(0, T, step=2)
        def _(tt):
            for kk in range(2):
                t = tt + kk

                @pl.when(t + 1 < T)
                def _():
                    gather_start(t + 1, 1 - kk)

                gather_wait(kk)

                for h in range(2):
                    @pl.when(t >= 1)
                    def _():
                        out_wait(h)

                    add_interleave(kk, h)

                    if kk == 0 and h == 0:
                        @pl.when((wid == 0) & (t == 0))
                        def _():
                            for lb in range(NB):
                                for v in range(128 // LANES):
                                    l = lb * 128 + v * LANES
                                    cv = cls_v[pl.ds(l, LANES)]
                                    for b in range(B):
                                        o = lb * B * 128 + b * 128 + v * LANES
                                        slab[h][pl.ds(o, LANES)] = cv

                    out_start(t, h)

        out_wait(0)
        out_wait(1)

        @pl.when(wid == NUM_WORKERS - 1)
        def _():
            pltpu.sync_copy(gidx_hbm.at[pl.ds(NUM_WORKERS * T * GI, 8)],
                            gidx_v.at[pl.ds(0, 8)])
            pltpu.async_copy(table_hbm.at[gidx_v.at[pl.ds(0, 8)]],
                             rows0.at[pl.ds(0, 8)], sg0).wait()
            pltpu.sync_copy(pidx_hbm.at[pl.ds(NUM_WORKERS * T * 8, 8)],
                            pidx_v.at[pl.ds(0, 8)])
            pltpu.async_copy(pos_hbm.at[pidx_v.at[pl.ds(0, CH)]],
                             posb0, sp0).wait()
            for lb in range(NB):
                for v in range(128 // LANES):
                    l = lb * 128 + v * LANES
                    pv = posb0[0, pl.ds(l, LANES)]
                    for b in range(B):
                        o = lb * B * 128 + b * 128 + v * LANES
                        slab0[pl.ds(o, LANES)] = (
                            rows0[b, pl.ds(l, LANES)] + pv)
            pltpu.sync_copy(slab0.at[pl.ds(0, B * D)],
                            out_hbm.at[pl.ds(S * B * D, B * D)])

    return sc_embed


def kernel(input, table, pos_embeds, cls):
    B, S = input.shape
    D = table.shape[1]
    SP = S + 1
    NB = D // 128
    S_PER_W = S // NUM_WORKERS
    sidx = jnp.zeros((B, SP), jnp.int32).at[:, 1:].set(input)
    gmain = (sidx[:, :S]
             .reshape(B, NUM_WORKERS, S_PER_W // CH, CH)
             .transpose(1, 2, 0, 3)
             .reshape(-1))
    gtail = jnp.concatenate([sidx[:, S], jnp.zeros((8 - B,), jnp.int32)])
    gidx = jnp.concatenate([gmain, gtail])
    n_items = S // CH
    pmain = jnp.clip(
        jnp.arange(n_items, dtype=jnp.int32)[:, None] * CH
        + jnp.arange(8, dtype=jnp.int32)[None, :] - 1,
        0, S - 1).reshape(-1)
    ptail = jnp.full((8,), S - 1, jnp.int32)
    pidx = jnp.concatenate([pmain, ptail])
    pos2d = pos_embeds.reshape(S, D)
    cls1d = cls.reshape(D)
    sc = _build_sc_kernel(B, S, D, NB)
    out_flat = sc(gidx, pidx, table, pos2d, cls1d)
    return (out_flat.reshape(SP, NB, B, 128)
            .transpose(2, 0, 1, 3)
            .reshape(B, SP, D))
